# baseline (speedup 1.0000x reference)
import functools

import jax
import jax.numpy as jnp
from jax import lax
from jax.experimental import pallas as pl
from jax.experimental.pallas import tpu as pltpu
from jax.experimental.pallas import tpu_sc as plsc

N_PER = 9000
MAX_DEG = 10
N_NODES = N_PER * (MAX_DEG + 1)
D = 128
F = 128

_C = 72
_NCH = N_PER // _C
_NB = 4
_TRASH = N_PER * MAX_DEG
_REL_ROWS = 91000
_DEG_BASE = {d: N_PER * (d * (d - 1) // 2) for d in range(1, MAX_DEG + 1)}
_IDX_PAD = 8192


_NBUF = 3
_AHEAD = 2


@functools.lru_cache(maxsize=None)
def _sc_gather_sum():
    info = plsc.get_sparse_core_info()
    nc, ns = info.num_cores, info.num_subcores
    nw = nc * ns
    assert nw * _NB >= _NCH
    mesh = plsc.VectorSubcoreMesh(core_axis_name="c", subcore_axis_name="s")
    max_idx = _NB * _C * MAX_DEG
    scratch = (
        [pltpu.VMEM((max_idx,), jnp.int32) for _ in range(2)]
        + [pltpu.VMEM((_C, D), jnp.float32) for _ in range(_NBUF)]
        + [pltpu.SemaphoreType.DMA for _ in range(_NBUF)]
        + [pltpu.SemaphoreType.DMA for _ in range(_NBUF)]
    )

    @functools.partial(
        pl.kernel,
        out_type=jax.ShapeDtypeStruct((_REL_ROWS, D), jnp.float32),
        mesh=mesh,
        scratch_types=scratch,
    )
    def sc_k(atoms_hbm, idx_hbm, rel_hbm, *refs):
        idx_bufs = refs[0:2]
        gbs = refs[2:2 + _NBUF]
        gsems = refs[2 + _NBUF:2 + 2 * _NBUF]
        ssems = refs[2 + 2 * _NBUF:2 + 3 * _NBUF]
        wid = lax.axis_index("s") * nc + lax.axis_index("c")
        my0 = wid * _NB
        zero = jnp.zeros((16,), jnp.float32)

        chain = [(d, j) for d in range(1, MAX_DEG + 1) for j in range(_NB)]
        n_chunks = len(chain)

        def idx_load(d):
            l_chunk = _C * d
            off = pl.multiple_of(_DEG_BASE[d] + my0 * l_chunk, 8)
            pltpu.sync_copy(idx_hbm.at[pl.ds(off, _NB * l_chunk)],
                            idx_bufs[(d - 1) % 2].at[pl.ds(0, _NB * l_chunk)])

        def zero_fire(pos):
            d, j = chain[pos]
            l_chunk = _C * d
            gb = gbs[pos % _NBUF]
            if d > 1:
                def zrow(r, _, gb=gb):
                    for c in range(D // 16):
                        gb[r, pl.ds(c * 16, 16)] = zero
                    return 0
                lax.fori_loop(0, _C, zrow, 0)
            for g in range(d):
                pltpu.async_copy(
                    atoms_hbm.at[
                        idx_bufs[(d - 1) % 2].at[
                            pl.ds(j * l_chunk + g * _C, _C)]],
                    gb,
                    gsems[pos % _NBUF],
                    add=(d > 1),
                )

        def consume(pos):
            d, j = chain[pos]
            gb = gbs[pos % _NBUF]
            for _ in range(d):
                pltpu.make_async_copy(
                    atoms_hbm.at[pl.ds(0, _C)], gb,
                    gsems[pos % _NBUF]).wait()
            k = my0 + j
            row_off = jnp.where(k < _NCH, (d - 1) * N_PER + k * _C, _TRASH)
            row_off = pl.multiple_of(row_off, 8)
            pltpu.sync_copy(gb, rel_hbm.at[pl.ds(row_off, _C)])

        idx_load(1)
        for pos in range(_AHEAD):
            if chain[pos][1] == 0 and chain[pos][0] > 1:
                idx_load(chain[pos][0])
            zero_fire(pos)
        for i in range(n_chunks):
            consume(i)
            n = i + _AHEAD
            if n < n_chunks:
                if chain[n][1] == 0:
                    idx_load(chain[n][0])
                zero_fire(n)

    return sc_k


def _tc_affine(rel, atoms, wr, ws, bb):
    blk = 1000
    n_blocks = N_NODES // blk
    per_deg = N_PER // blk

    def body(xr_ref, xs_ref, wr_ref, ws_ref, b_ref, o_ref):
        acc = jnp.dot(xr_ref[...], wr_ref[0], preferred_element_type=jnp.float32)
        acc = acc + jnp.dot(xs_ref[...], ws_ref[0], preferred_element_type=jnp.float32)
        o_ref[...] = jnp.maximum(acc + b_ref[0], 0.0)

    return pl.pallas_call(
        body,
        grid=(n_blocks,),
        in_specs=[
            pl.BlockSpec((blk, D), lambda i: (jnp.maximum(i - per_deg, 0), 0)),
            pl.BlockSpec((blk, D), lambda i: (i, 0)),
            pl.BlockSpec((1, D, F), lambda i: (i // per_deg, 0, 0)),
            pl.BlockSpec((1, D, F), lambda i: (i // per_deg, 0, 0)),
            pl.BlockSpec((1, 1, F), lambda i: (i // per_deg, 0, 0)),
        ],
        out_specs=pl.BlockSpec((blk, F), lambda i: (i, 0)),
        out_shape=jax.ShapeDtypeStruct((N_NODES, F), jnp.float32),
    )(rel, atoms, wr, ws, bb)


def kernel(atom_features, deg_slice, membership, deg_adj_1, deg_adj_2,
           deg_adj_3, deg_adj_4, deg_adj_5, deg_adj_6, deg_adj_7, deg_adj_8,
           deg_adj_9, deg_adj_10, W, b):
    adjs = [deg_adj_1, deg_adj_2, deg_adj_3, deg_adj_4, deg_adj_5, deg_adj_6,
            deg_adj_7, deg_adj_8, deg_adj_9, deg_adj_10]
    idx = jnp.concatenate(
        [a.reshape(_NCH, _C, d + 1).transpose(0, 2, 1).reshape(-1)
         for d, a in enumerate(adjs)]
        + [jnp.zeros((_IDX_PAD,), jnp.int32)]
    )
    rel = _sc_gather_sum()(atom_features, idx)
    wr = jnp.concatenate([jnp.zeros((1, D, F), W.dtype), W[0:20:2]], axis=0)
    ws = jnp.concatenate([W[20:21], W[1:20:2]], axis=0)
    bb = jnp.concatenate([b[20:21], b[0:20:2] + b[1:20:2]], axis=0)
    bb = bb.reshape(MAX_DEG + 1, 1, F)
    return _tc_affine(rel, atom_features, wr, ws, bb)


# speedup vs baseline: 1.3167x; 1.3167x over previous
import functools

import jax
import jax.numpy as jnp
from jax import lax
from jax.experimental import pallas as pl
from jax.experimental.pallas import tpu as pltpu
from jax.experimental.pallas import tpu_sc as plsc

N_PER = 9000
MAX_DEG = 10
N_NODES = N_PER * (MAX_DEG + 1)
D = 128
F = 128

_C = 72
_NCH = N_PER // _C
_DEG_BASE = {d: N_PER * (d * (d - 1) // 2) for d in range(1, MAX_DEG + 1)}
_IDX_PAD = 8192


@functools.lru_cache(maxsize=None)
def _sc_gather_sum():
    info = plsc.get_sparse_core_info()
    nc, ns = info.num_cores, info.num_subcores
    nw = nc * ns
    niter = -(-_NCH // nw)
    mesh = plsc.VectorSubcoreMesh(core_axis_name="c", subcore_axis_name="s")
    max_idx = niter * _C * MAX_DEG
    scratch = [
        pltpu.VMEM((max_idx,), jnp.int32),
        pltpu.VMEM((_C, D), jnp.float32),
        pltpu.VMEM((_C, D), jnp.float32),
        pltpu.SemaphoreType.DMA,
        pltpu.SemaphoreType.DMA,
    ]

    @functools.partial(
        pl.kernel,
        out_type=jax.ShapeDtypeStruct((N_PER * MAX_DEG, D), jnp.float32),
        mesh=mesh,
        scratch_types=scratch,
    )
    def sc_k(atoms_hbm, idx_hbm, rel_hbm, idx_all, gb0, gb1, sem0, sem1):
        wid = lax.axis_index("s") * nc + lax.axis_index("c")
        zero = jnp.zeros((16,), jnp.float32)

        for d in range(1, MAX_DEG + 1):
            l_chunk = _C * d
            base = _DEG_BASE[d]
            out_base = (d - 1) * N_PER
            my0 = wid * niter
            lim = jnp.minimum(_NCH - my0, niter)

            idx_off = pl.multiple_of(base + my0 * l_chunk, 8)
            pltpu.sync_copy(idx_hbm.at[pl.ds(idx_off, niter * l_chunk)],
                            idx_all.at[pl.ds(0, niter * l_chunk)])

            def prep_fire(j, gb, sem, d=d, l_chunk=l_chunk):
                if d > 1:
                    def zrow(r, _):
                        for c in range(D // 16):
                            gb[r, pl.ds(c * 16, 16)] = zero
                        return 0
                    lax.fori_loop(0, _C, zrow, 0)
                for g in range(d):
                    pltpu.async_copy(
                        atoms_hbm.at[idx_all.at[pl.ds(j * l_chunk + g * _C, _C)]],
                        gb,
                        sem,
                        add=(d > 1),
                    )

            def consume(j, gb, sem, d=d, out_base=out_base, my0=my0):
                for _ in range(d):
                    pltpu.make_async_copy(
                        atoms_hbm.at[pl.ds(0, _C)], gb, sem
                    ).wait()
                row_off = pl.multiple_of(out_base + (my0 + j) * _C, 8)
                pltpu.sync_copy(gb, rel_hbm.at[pl.ds(row_off, _C)])

            @pl.when(0 < lim)
            def _():
                prep_fire(0, gb0, sem0)

            def pair_body(t, _):
                j0 = 2 * t
                j1 = j0 + 1
                j2 = j0 + 2

                @pl.when(j1 < lim)
                def _():
                    prep_fire(j1, gb1, sem1)

                @pl.when(j0 < lim)
                def _():
                    consume(j0, gb0, sem0)

                @pl.when(j2 < lim)
                def _():
                    prep_fire(j2, gb0, sem0)

                @pl.when(j1 < lim)
                def _():
                    consume(j1, gb1, sem1)

                return 0

            lax.fori_loop(0, -(-niter // 2), pair_body, 0)

    return sc_k


def _tc_affine(rel, atoms, wr, ws, bb):
    blk = 1000
    n_blocks = N_NODES // blk
    per_deg = N_PER // blk

    def body(xr_ref, xs_ref, wr_ref, ws_ref, b_ref, o_ref):
        acc = jnp.dot(xr_ref[...], wr_ref[0], preferred_element_type=jnp.float32)
        acc = acc + jnp.dot(xs_ref[...], ws_ref[0], preferred_element_type=jnp.float32)
        o_ref[...] = jnp.maximum(acc + b_ref[0], 0.0)

    return pl.pallas_call(
        body,
        grid=(n_blocks,),
        in_specs=[
            pl.BlockSpec((blk, D), lambda i: (jnp.maximum(i - per_deg, 0), 0)),
            pl.BlockSpec((blk, D), lambda i: (i, 0)),
            pl.BlockSpec((1, D, F), lambda i: (i // per_deg, 0, 0)),
            pl.BlockSpec((1, D, F), lambda i: (i // per_deg, 0, 0)),
            pl.BlockSpec((1, 1, F), lambda i: (i // per_deg, 0, 0)),
        ],
        out_specs=pl.BlockSpec((blk, F), lambda i: (i, 0)),
        out_shape=jax.ShapeDtypeStruct((N_NODES, F), jnp.float32),
    )(rel, atoms, wr, ws, bb)


def kernel(atom_features, deg_slice, membership, deg_adj_1, deg_adj_2,
           deg_adj_3, deg_adj_4, deg_adj_5, deg_adj_6, deg_adj_7, deg_adj_8,
           deg_adj_9, deg_adj_10, W, b):
    adjs = [deg_adj_1, deg_adj_2, deg_adj_3, deg_adj_4, deg_adj_5, deg_adj_6,
            deg_adj_7, deg_adj_8, deg_adj_9, deg_adj_10]
    idx = jnp.concatenate(
        [a.reshape(_NCH, _C, d + 1).transpose(0, 2, 1).reshape(-1)
         for d, a in enumerate(adjs)]
        + [jnp.zeros((_IDX_PAD,), jnp.int32)]
    )
    rel = _sc_gather_sum()(atom_features, idx)
    wr = jnp.concatenate([jnp.zeros((1, D, F), W.dtype), W[0:20:2]], axis=0)
    ws = jnp.concatenate([W[20:21], W[1:20:2]], axis=0)
    bb = jnp.concatenate([b[20:21], b[0:20:2] + b[1:20:2]], axis=0)
    bb = bb.reshape(MAX_DEG + 1, 1, F)
    return _tc_affine(rel, atom_features, wr, ws, bb)
